# baseline (speedup 1.0000x reference)
import functools

import jax
import jax.numpy as jnp
from jax import lax
from jax.experimental import pallas as pl
from jax.experimental.pallas import tpu as pltpu
from jax.experimental.pallas import tpu_sc as plsc

NUM_ENT = 40000
NUM_TIM = 10000
D = 128
NB = 4
NR = 200
N = 10000
E = 160000
B = 128
TU = 100

NC = 2
NS = 16
NW = NC * NS

E_PAD = 163840
EW = E_PAD // NW
PREP_C = 128
LAY_C = 32
N_PAD = 10240
XW = N_PAD // NW
N_ACC = 10112
NROW = N_ACC // NS
TUP = 112
G_PAD = 10240

_mesh = plsc.VectorSubcoreMesh(
    core_axis_name="c", subcore_axis_name="s", num_cores=NC, num_subcores=NS)


def _wid():
    return lax.axis_index("c") * NS + lax.axis_index("s")


def _splat(val):
    return jnp.full((16,), val, jnp.int32)


NCH_P = EW // PREP_C


@functools.partial(
    pl.kernel,
    mesh=_mesh,
    compiler_params=pltpu.CompilerParams(needs_layout_passes=False),
    out_type=[
        jax.ShapeDtypeStruct((N_PAD, D), jnp.float32),
        jax.ShapeDtypeStruct((E_PAD * NB,), jnp.float32),
        jax.ShapeDtypeStruct((E_PAD * NB,), jnp.float32),
        jax.ShapeDtypeStruct((TUP, D), jnp.float32),
    ],
    scratch_types=[
        pltpu.VMEM((XW,), jnp.int32),
        pltpu.VMEM((XW, D), jnp.float32),
        pltpu.VMEM((EW,), jnp.int32),
        pltpu.VMEM((EW,), jnp.int32),
        pltpu.VMEM((EW,), jnp.int32),
        pltpu.VMEM((EW,), jnp.int32),
        pltpu.VMEM((EW,), jnp.float32),
        pltpu.VMEM((EW,), jnp.int32),
        pltpu.VMEM((EW,), jnp.int32),
        [pltpu.VMEM((PREP_C,), jnp.float32) for _ in range(2)],
        [pltpu.VMEM((PREP_C,), jnp.float32) for _ in range(2)],
        pltpu.VMEM((NR * NB,), jnp.float32),
        pltpu.VMEM((NR * NB,), jnp.float32),
        [pltpu.VMEM((PREP_C * NB,), jnp.float32) for _ in range(2)],
        [pltpu.VMEM((PREP_C * NB,), jnp.float32) for _ in range(2)],
        pltpu.VMEM((TUP,), jnp.int32),
        pltpu.VMEM((TUP, D), jnp.float32),
        pltpu.SemaphoreType.DMA,
        pltpu.SemaphoreType.DMA,
        [pltpu.SemaphoreType.DMA for _ in range(2)],
        [pltpu.SemaphoreType.DMA for _ in range(2)],
    ],
)
def _sc_prep(entity_hbm, emb_hbm, gflat_hbm, ebi_hbm, st_hbm, et_hbm,
             type_hbm, norm_hbm, comp1_hbm, comp2_hbm, ut_hbm,
             x_out, w1_out, w2_out, temb_out,
             idx_v, rows_v, ebi_all, st_all, et_all, type_all, norm_all,
             ist_all, iet_all, gst, get, c1_v, c2_v, w1v, w2v, ut_v,
             trows_v, psem, xsem, gsem, osem):
    wid = _wid()
    esl = pl.ds(wid * EW, EW)

    cps = [pltpu.async_copy(ebi_hbm.at[esl], ebi_all, psem),
           pltpu.async_copy(st_hbm.at[esl], st_all, psem),
           pltpu.async_copy(et_hbm.at[esl], et_all, psem),
           pltpu.async_copy(type_hbm.at[esl], type_all, psem),
           pltpu.async_copy(norm_hbm.at[esl], norm_all, psem),
           pltpu.async_copy(comp1_hbm, c1_v, psem),
           pltpu.async_copy(comp2_hbm, c2_v, psem),
           pltpu.async_copy(entity_hbm.at[pl.ds(wid * XW, XW)], idx_v,
                            psem)]
    for cp in cps:
        cp.wait()

    pltpu.async_copy(emb_hbm.at[idx_v.at[pl.ds(0, 128)]],
                     rows_v.at[pl.ds(0, 128)], xsem)
    pltpu.async_copy(emb_hbm.at[idx_v.at[pl.ds(128, 128)]],
                     rows_v.at[pl.ds(128, 128)], xsem)
    pltpu.async_copy(emb_hbm.at[idx_v.at[pl.ds(256, 64)]],
                     rows_v.at[pl.ds(256, 64)], xsem)

    @pl.when(wid == 0)
    def _():
        pltpu.sync_copy(ut_hbm, ut_v)
        for k in range(TUP // 16):
            sl = pl.ds(k * 16, 16)
            ut_v[sl] = ut_v[sl] + NUM_ENT
        pltpu.async_copy(emb_hbm.at[ut_v], trows_v, psem).wait()
        pltpu.sync_copy(trows_v, temb_out)

    @plsc.parallel_loop(0, EW // 16, 1, unroll=4)
    def _idx_body(i):
        sl = pl.ds(i * 16, 16)
        bt = ebi_all[sl] * G_PAD
        ist_all[sl] = bt + st_all[sl]
        iet_all[sl] = bt + et_all[sl]

    def istsl(c):
        return ist_all.at[pl.ds(c * PREP_C, PREP_C)]

    def ietsl(c):
        return iet_all.at[pl.ds(c * PREP_C, PREP_C)]

    def w1sl(c):
        return w1_out.at[pl.ds((wid * EW + c * PREP_C) * NB, PREP_C * NB)]

    def w2sl(c):
        return w2_out.at[pl.ds((wid * EW + c * PREP_C) * NB, PREP_C * NB)]

    for j in range(2):
        pltpu.async_copy(gflat_hbm.at[istsl(j)], gst[j], gsem[j])
        pltpu.async_copy(gflat_hbm.at[ietsl(j)], get[j], gsem[j])

    lanes = lax.iota(jnp.int32, 16)

    def wchunk(c, b, last):
        pltpu.make_async_copy(gflat_hbm.at[istsl(c)], gst[b], gsem[b]).wait()
        pltpu.make_async_copy(gflat_hbm.at[ietsl(c)], get[b], gsem[b]).wait()
        @pl.when(c >= 2)
        def _():
            pltpu.make_async_copy(w1v[b], w1sl(c), osem[b]).wait()
            pltpu.make_async_copy(w2v[b], w2sl(c), osem[b]).wait()
        e0 = c * PREP_C
        for k in range(PREP_C // 16):
            sl = pl.ds(k * 16, 16)
            asl = pl.ds(e0 + k * 16, 16)
            z = (gst[b][sl] + get[b][sl]) * 0.5
            gate = 1.0 / (1.0 + jnp.exp(-z))
            sval = gate * norm_all[asl]
            tk = type_all[asl] * NB
            for bi in range(NB):
                cidx = lanes * NB + (k * 16 * NB + bi)
                tb = tk + bi
                plsc.store_scatter(w1v[b], [cidx],
                                   plsc.load_gather(c1_v, [tb]) * sval)
                plsc.store_scatter(w2v[b], [cidx],
                                   plsc.load_gather(c2_v, [tb]) * sval)
        pltpu.async_copy(w1v[b], w1sl(c), osem[b])
        pltpu.async_copy(w2v[b], w2sl(c), osem[b])
        if not last:
            pltpu.async_copy(gflat_hbm.at[istsl(c + 2)], gst[b], gsem[b])
            pltpu.async_copy(gflat_hbm.at[ietsl(c + 2)], get[b], gsem[b])

    def pair_body(q, _):
        wchunk(2 * q, 0, False)
        wchunk(2 * q + 1, 1, False)
        return 0

    lax.fori_loop(0, NCH_P // 2 - 1, pair_body, 0)
    wchunk(jnp.int32(NCH_P - 2), 0, True)
    wchunk(jnp.int32(NCH_P - 1), 1, True)
    for b in range(2):
        pltpu.make_async_copy(w1v[b], w1sl(jnp.int32(NCH_P - 2 + b)),
                              osem[b]).wait()
        pltpu.make_async_copy(w2v[b], w2sl(jnp.int32(NCH_P - 2 + b)),
                              osem[b]).wait()

    pltpu.make_async_copy(emb_hbm.at[idx_v.at[pl.ds(0, 128)]],
                          rows_v.at[pl.ds(0, 128)], xsem).wait()
    pltpu.make_async_copy(emb_hbm.at[idx_v.at[pl.ds(128, 128)]],
                          rows_v.at[pl.ds(128, 128)], xsem).wait()
    pltpu.make_async_copy(emb_hbm.at[idx_v.at[pl.ds(256, 64)]],
                          rows_v.at[pl.ds(256, 64)], xsem).wait()
    pltpu.sync_copy(rows_v, x_out.at[pl.ds(wid * XW, XW)])


@functools.partial(
    pl.kernel,
    mesh=_mesh,
    compiler_params=pltpu.CompilerParams(needs_layout_passes=False),
    out_type=jax.ShapeDtypeStruct((NC, N_ACC, D), jnp.float32),
    scratch_types=[
        pltpu.VMEM((EW,), jnp.int32),
        [pltpu.VMEM((LAY_C,), jnp.int32) for _ in range(4)],
        [pltpu.VMEM((LAY_C * NB,), jnp.float32) for _ in range(4)],
        [pltpu.VMEM((LAY_C, NB * D // 2), jnp.int32) for _ in range(2)],
        [pltpu.VMEM((LAY_C, D), jnp.float32) for _ in range(2)],
        pltpu.VMEM_SHARED((N_ACC, D), jnp.float32),
        [pltpu.SemaphoreType.DMA for _ in range(2)],
        [pltpu.SemaphoreType.DMA for _ in range(4)],
        [pltpu.SemaphoreType.DMA for _ in range(2)],
        pltpu.SemaphoreType.DMA,
    ],
)
def _sc_layer(xb_hbm, src_hbm, dst_hbm, w_hbm, zr_hbm, part_out,
              sidx_all, didx, w, rows, msg, acc, gsem, wsem, ssem, psem):
    cid = lax.axis_index("c")
    sid = lax.axis_index("s")
    wid = cid * NS + sid
    nch = EW // LAY_C
    wstride = LAY_C * NB

    def wslice(ch):
        ch = jnp.minimum(ch, nch - 1)
        return w_hbm.at[pl.ds(wid * EW * NB + ch * wstride, wstride)]

    def dslice(ch):
        ch = jnp.minimum(ch, nch - 1)
        return dst_hbm.at[pl.ds(wid * EW + ch * LAY_C, LAY_C)]

    def sidx(ch):
        ch = jnp.minimum(ch, nch - 1)
        return sidx_all.at[pl.ds(ch * LAY_C, LAY_C)]

    c0 = pltpu.async_copy(src_hbm.at[pl.ds(wid * EW, EW)], sidx_all, psem)
    pltpu.sync_copy(zr_hbm, acc.at[pl.ds(sid * NROW, NROW)])
    c0.wait()
    plsc.subcore_barrier()

    for j in range(2):
        pltpu.async_copy(wslice(j), w[j], wsem[j])
        pltpu.async_copy(dslice(j), didx[j], wsem[j])
    pltpu.async_copy(xb_hbm.at[sidx(0)], rows[0], gsem[0])

    def compute(rows_v, w_v, msg_v):
        @plsc.parallel_loop(0, LAY_C, 1, unroll=4)
        def edge_body(c):
            cb = c * NB
            wb = [plsc.load_gather(w_v, [_splat(cb + bi)])
                  for bi in range(NB)]
            for kb in range(D // 32):
                parts = [plsc.unpack(
                    plsc.bitcast(rows_v[c, pl.ds(bi * (D // 2) + kb * 16,
                                                 16)], jnp.bfloat16),
                    format=plsc.PackFormat.INTERLEAVED)
                         for bi in range(NB)]
                for h in range(2):
                    m = wb[0] * parts[0][h]
                    m = m + wb[1] * parts[1][h]
                    m = m + wb[2] * parts[2][h]
                    m = m + wb[3] * parts[3][h]
                    msg_v[c, pl.ds(kb * 32 + h * 16, 16)] = m

    def half(c, j):
        mb = j % 2
        db = j % 4
        db2 = (j + 2) % 4
        pltpu.make_async_copy(xb_hbm.at[sidx(c)], rows[mb], gsem[mb]).wait()
        @pl.when(c >= 2)
        def _():
            pltpu.make_async_copy(msg[mb], acc.at[didx[db2]],
                                  ssem[mb]).wait()
        @pl.when(c + 1 < nch)
        def _():
            pltpu.async_copy(xb_hbm.at[sidx(c + 1)], rows[1 - mb],
                             gsem[1 - mb])
        @pl.when(c + 2 < nch)
        def _():
            pltpu.async_copy(wslice(c + 2), w[db2], wsem[db2])
            pltpu.async_copy(dslice(c + 2), didx[db2], wsem[db2])
        pltpu.make_async_copy(wslice(c), w[db], wsem[db]).wait()
        pltpu.make_async_copy(dslice(c), didx[db], wsem[db]).wait()
        compute(rows[mb], w[db], msg[mb])
        pltpu.async_copy(msg[mb], acc.at[didx[db]], ssem[mb], add=True)

    def quad_body(q, _):
        for j in range(4):
            half(4 * q + j, j)
        return 0

    lax.fori_loop(0, nch // 4, quad_body, 0)
    pltpu.make_async_copy(msg[0], acc.at[didx[2]], ssem[0]).wait()
    pltpu.make_async_copy(msg[1], acc.at[didx[3]], ssem[1]).wait()
    plsc.subcore_barrier()
    rsl = pl.ds(sid * NROW, NROW)
    pltpu.sync_copy(acc.at[rsl], part_out.at[cid, rsl])


EPW = 56
TPW = 24
PW = 16
BPW = B // PW


@functools.partial(
    pl.kernel,
    mesh=_mesh,
    compiler_params=pltpu.CompilerParams(needs_layout_passes=False),
    out_type=[
        jax.ShapeDtypeStruct((B, D), jnp.float32),
        jax.ShapeDtypeStruct((B, D), jnp.float32),
    ],
    scratch_types=[
        pltpu.VMEM((2 * EPW,), jnp.int32),
        pltpu.VMEM((2 * EPW,), jnp.int32),
        pltpu.VMEM((2 * EPW,), jnp.int32),
        pltpu.VMEM((2 * EPW,), jnp.int32),
        pltpu.VMEM((2 * EPW, D), jnp.float32),
        pltpu.VMEM((2 * EPW, D), jnp.float32),
        pltpu.VMEM((2 * EPW, D), jnp.float32),
        pltpu.VMEM((2 * EPW, D), jnp.float32),
        pltpu.VMEM((BPW * TPW,), jnp.int32),
        pltpu.VMEM((BPW * TPW, D), jnp.float32),
        pltpu.VMEM((BPW, D), jnp.float32),
        pltpu.VMEM((BPW, D), jnp.float32),
        pltpu.SemaphoreType.DMA,
    ],
)
def _sc_pool(h2_hbm, temb_hbm, epool_hbm, tpool_hbm, pe_out, pt_out,
             eidx_a, eidx_b, eidx_c, eidx_d, erows_a, erows_b, erows_c,
             erows_d, tidx_v, trows_v, pe_v, pt_v, sem):
    cid = lax.axis_index("c")
    sid = lax.axis_index("s")
    @pl.when(cid == 0)
    def _():
        wid = sid
        ne = BPW * EPW
        quarter = ne // 4
        eidx = [eidx_a, eidx_b, eidx_c, eidx_d]
        erows = [erows_a, erows_b, erows_c, erows_d]
        for qq in range(4):
            pltpu.sync_copy(
                epool_hbm.at[pl.ds(wid * ne + qq * quarter, quarter)],
                eidx[qq])
        for qq in range(4):
            pltpu.async_copy(h2_hbm.at[eidx[qq]], erows[qq], sem)
        for qq in range(4):
            pltpu.make_async_copy(h2_hbm.at[eidx[qq]], erows[qq],
                                  sem).wait()

        for j in range(BPW):
            ebuf = erows[j // 2]
            joff = (j % 2) * EPW

            def eadd(r, carry):
                return tuple(carry[k] + ebuf[joff + r, pl.ds(k * 16, 16)]
                             for k in range(D // 16))
            acc8 = lax.fori_loop(
                0, 50, eadd, tuple(jnp.zeros((16,), jnp.float32)
                                   for _ in range(D // 16)))
            for k in range(D // 16):
                pe_v[j, pl.ds(k * 16, 16)] = acc8[k] * (1.0 / 50.0)

        bsl = pl.ds(wid * BPW, BPW)
        pltpu.sync_copy(pe_v, pe_out.at[bsl])

    @pl.when(cid == 1)
    def _():
        wid = sid
        nt = BPW * TPW
        pltpu.sync_copy(tpool_hbm.at[pl.ds(wid * nt, nt)], tidx_v)
        pltpu.async_copy(temb_hbm.at[tidx_v], trows_v, sem).wait()

        for j in range(BPW):
            def tadd(r, carry):
                return tuple(carry[k] + trows_v[j * TPW + r,
                                                pl.ds(k * 16, 16)]
                             for k in range(D // 16))
            acc8t = lax.fori_loop(
                0, 20, tadd, tuple(jnp.zeros((16,), jnp.float32)
                                   for _ in range(D // 16)))
            for k in range(D // 16):
                pt_v[j, pl.ds(k * 16, 16)] = acc8t[k] * (1.0 / 20.0)

        bsl = pl.ds(wid * BPW, BPW)
        pltpu.sync_copy(pt_v, pt_out.at[bsl])


def _dot(a, b):
    return jax.lax.dot_general(a, b, (((1,), (0,)), ((), ())),
                               preferred_element_type=jnp.float32)


def _dot_t(a, b):
    return jax.lax.dot_general(a, b, (((1,), (1,)), ((), ())),
                               preferred_element_type=jnp.float32)


def _tc_prep_body(qb_ref, wl_ref, bl_ref, wa_ref, ba_ref, wp_ref, bp_ref,
                  q_ref, wq_ref, p_ref):
    qb = qb_ref[...]
    q_ref[...] = _dot(qb, wl_ref[...]) + bl_ref[...]
    wq_ref[...] = _dot(qb, wa_ref[...]) + ba_ref[...]
    p_ref[...] = jax.nn.sigmoid(_dot(qb, wp_ref[...]) + bp_ref[...])


def _tc_prep(qb, wl, bl, wa, ba, wpb, bpb):
    return pl.pallas_call(
        _tc_prep_body,
        out_shape=[jax.ShapeDtypeStruct((B, D), jnp.float32),
                   jax.ShapeDtypeStruct((B, D), jnp.float32),
                   jax.ShapeDtypeStruct((B, D), jnp.float32)],
    )(qb, wl, bl, wa, ba, wpb, bpb)


GT = 1280


def _tc_g_body(wq_ref, t_ref, g_ref):
    g_ref[...] = _dot_t(wq_ref[...], t_ref[...])


def _tc_g(wq, t):
    return pl.pallas_call(
        _tc_g_body,
        grid=(G_PAD // GT,),
        in_specs=[pl.BlockSpec((B, D), lambda i: (0, 0)),
                  pl.BlockSpec((GT, D), lambda i: (i, 0))],
        out_specs=pl.BlockSpec((B, GT), lambda i: (0, i)),
        out_shape=jax.ShapeDtypeStruct((B, G_PAD), jnp.float32),
    )(wq, t)


XT = 400


def _tc_xb_body(x_ref, bc_ref, xb_ref):
    xb_ref[...] = _dot(x_ref[...], bc_ref[...]).astype(jnp.bfloat16)


def _tc_xb(x, bcat):
    return pl.pallas_call(
        _tc_xb_body,
        grid=(N // XT,),
        in_specs=[pl.BlockSpec((XT, D), lambda i: (i, 0)),
                  pl.BlockSpec((D, NB * D), lambda i: (0, 0))],
        out_specs=pl.BlockSpec((XT, NB * D), lambda i: (i, 0)),
        out_shape=jax.ShapeDtypeStruct((N, NB * D), jnp.bfloat16),
    )(x, bcat)


def _tc_combine_body(part_ref, x_ref, ws_ref, b_ref, bc_ref,
                     h_ref, xb_ref):
    h = part_ref[0] + part_ref[1] + _dot(x_ref[...], ws_ref[...]) \
        + b_ref[0:1, :]
    h = jnp.maximum(h, 0.0)
    h_ref[...] = h
    xb_ref[...] = _dot(h, bc_ref[...]).astype(jnp.bfloat16)


def _tc_combine(part, x, ws, bvec, bcat):
    return pl.pallas_call(
        _tc_combine_body,
        grid=(N // XT,),
        in_specs=[pl.BlockSpec((NC, XT, D), lambda i: (0, i, 0)),
                  pl.BlockSpec((XT, D), lambda i: (i, 0)),
                  pl.BlockSpec((D, D), lambda i: (0, 0)),
                  pl.BlockSpec((8, D), lambda i: (0, 0)),
                  pl.BlockSpec((D, NB * D), lambda i: (0, 0))],
        out_specs=[pl.BlockSpec((XT, D), lambda i: (i, 0)),
                   pl.BlockSpec((XT, NB * D), lambda i: (i, 0))],
        out_shape=[jax.ShapeDtypeStruct((N, D), jnp.float32),
                   jax.ShapeDtypeStruct((N, NB * D), jnp.bfloat16)],
    )(part, x, ws, bvec, bcat)


def _tc_h2_body(part_ref, h_ref, ws_ref, b_ref, h2_ref):
    h2_ref[...] = part_ref[0] + part_ref[1] \
        + _dot(h_ref[...], ws_ref[...]) + b_ref[0:1, :]


def _tc_h2(part, h, ws, bvec):
    return pl.pallas_call(
        _tc_h2_body,
        grid=(N // XT,),
        in_specs=[pl.BlockSpec((NC, XT, D), lambda i: (0, i, 0)),
                  pl.BlockSpec((XT, D), lambda i: (i, 0)),
                  pl.BlockSpec((D, D), lambda i: (0, 0)),
                  pl.BlockSpec((8, D), lambda i: (0, 0))],
        out_specs=pl.BlockSpec((XT, D), lambda i: (i, 0)),
        out_shape=jax.ShapeDtypeStruct((N, D), jnp.float32),
    )(part, h, ws, bvec)


ST = 1280
NV = NUM_ENT + NUM_TIM


def _tc_score_body(q_ref, pe_ref, pt_ref, p_ref, emb_ref, out_ref):
    p = p_ref[...]
    pred = (q_ref[...] + pe_ref[...] * p + pt_ref[...] * (1.0 - p)) / 3.0
    nrm = jnp.sqrt(jnp.sum(pred * pred, axis=1, keepdims=True))
    pn = pred / (nrm + 1e-8)
    blk = emb_ref[...]
    nr = jnp.sqrt(jnp.sum(blk * blk, axis=1))
    s = _dot_t(pn, blk)
    out_ref[...] = s * (30.0 / (nr + 1e-8))[None, :]


def _tc_score(q, pe, pt, p, emb):
    bspec = pl.BlockSpec((B, D), lambda i: (0, 0))
    return pl.pallas_call(
        _tc_score_body,
        grid=((NV + ST - 1) // ST,),
        in_specs=[bspec, bspec, bspec, bspec,
                  pl.BlockSpec((ST, D), lambda i: (i, 0))],
        out_specs=pl.BlockSpec((B, ST), lambda i: (0, i)),
        out_shape=jax.ShapeDtypeStruct((B, NV), jnp.float32),
    )(q, pe, pt, p, emb)


def kernel(ques_emb_bert, entity, edge_index, edge_type, start_time,
           end_time, edge_norm, edge_batch_idx, uniq_times, ent_pool_idx,
           time_pool_idx, emb_table, basis1, comp1, w_self1, b1, basis2,
           comp2, w_self2, b2, w_lin, b_lin, w_attn, b_attn, w_p, b_p):
    i32 = jnp.int32
    entity = entity.astype(i32)
    edge_type = edge_type.astype(i32)
    start_time = start_time.astype(i32)
    end_time = end_time.astype(i32)
    edge_batch_idx = edge_batch_idx.astype(i32)
    uniq_times = uniq_times.astype(i32)

    wpb = jnp.broadcast_to(w_p, (768, D))
    bpb = jnp.broadcast_to(b_p, (D,))
    q_emb, wq, p = _tc_prep(ques_emb_bert, w_lin, b_lin, w_attn, b_attn,
                            wpb, bpb)
    tpad = jnp.concatenate(
        [emb_table[NUM_ENT:], jnp.zeros((G_PAD - NUM_TIM, D), jnp.float32)])
    g = _tc_g(wq, tpad)
    gflat = g.reshape(-1)

    pad_e = E_PAD - E
    src = jnp.concatenate([edge_index[0].astype(i32), jnp.zeros(pad_e, i32)])
    dst = jnp.concatenate([edge_index[1].astype(i32), jnp.zeros(pad_e, i32)])
    ebi_p = jnp.concatenate([edge_batch_idx, jnp.zeros(pad_e, i32)])
    st_p = jnp.concatenate([start_time, jnp.zeros(pad_e, i32)])
    et_p = jnp.concatenate([end_time, jnp.zeros(pad_e, i32)])
    ty_p = jnp.concatenate([edge_type, jnp.zeros(pad_e, i32)])
    nm_p = jnp.concatenate([edge_norm, jnp.zeros(pad_e, jnp.float32)])
    ent_p = jnp.concatenate([entity, jnp.zeros(N_PAD - N, i32)])
    ut_p = jnp.concatenate([uniq_times, jnp.zeros(TUP - TU, i32)])

    x_pad, w1, w2, t_emb = _sc_prep(ent_p, emb_table, gflat, ebi_p, st_p,
                                    et_p, ty_p, nm_p, comp1.reshape(-1),
                                    comp2.reshape(-1), ut_p)
    x = x_pad[:N]

    zr = jnp.zeros((NROW, D), jnp.float32)
    b1b = jnp.broadcast_to(b1, (8, D))
    b2b = jnp.broadcast_to(b2, (8, D))
    pblk = jnp.arange(32).reshape(2, 16).T.reshape(-1)
    perm = (jnp.arange(NB * D) // 32) * 32 + pblk[jnp.arange(NB * D) % 32]
    b1cat = jnp.transpose(basis1, (1, 0, 2)).reshape(D, NB * D)[:, perm]
    b2cat = jnp.transpose(basis2, (1, 0, 2)).reshape(D, NB * D)[:, perm]

    xb1 = _tc_xb(x, b1cat)
    xb1_i = lax.bitcast_convert_type(
        xb1.reshape(N, NB * D // 2, 2), jnp.int32)
    part1 = _sc_layer(xb1_i, src, dst, w1, zr)
    h1, xb2 = _tc_combine(part1, x, w_self1, b1b, b2cat)
    xb2_i = lax.bitcast_convert_type(
        xb2.reshape(N, NB * D // 2, 2), jnp.int32)
    part2 = _sc_layer(xb2_i, src, dst, w2, zr)
    h2 = _tc_h2(part2, h1, w_self2, b2b)

    ep = jnp.pad(ent_pool_idx.astype(i32), ((0, 0), (0, EPW - 50)))
    tp = jnp.pad(time_pool_idx.astype(i32), ((0, 0), (0, TPW - 20)))
    pe, pt = _sc_pool(h2, t_emb, ep.reshape(-1), tp.reshape(-1))

    return _tc_score(q_emb, pe, pt, p, emb_table)


# speedup vs baseline: 1.2394x; 1.2394x over previous
import functools

import jax
import jax.numpy as jnp
from jax import lax
from jax.experimental import pallas as pl
from jax.experimental.pallas import tpu as pltpu
from jax.experimental.pallas import tpu_sc as plsc

NUM_ENT = 40000
NUM_TIM = 10000
D = 128
NB = 4
NR = 200
N = 10000
E = 160000
B = 128
TU = 100

NC = 2
NS = 16
NW = NC * NS

E_PAD = 163840
EW = E_PAD // NW
PREP_C = 128
LAY_C = 32
N_PAD = 10240
XW = N_PAD // NW
N_ACC = 10112
NROW = N_ACC // NS
TUP = 112
G_PAD = 10240

_mesh = plsc.VectorSubcoreMesh(
    core_axis_name="c", subcore_axis_name="s", num_cores=NC, num_subcores=NS)


def _wid():
    return lax.axis_index("c") * NS + lax.axis_index("s")


def _splat(val):
    return jnp.full((16,), val, jnp.int32)


NCH_P = EW // PREP_C


@functools.partial(
    pl.kernel,
    mesh=_mesh,
    compiler_params=pltpu.CompilerParams(needs_layout_passes=False),
    out_type=[
        jax.ShapeDtypeStruct((N_PAD, D), jnp.float32),
        jax.ShapeDtypeStruct((E_PAD * NB,), jnp.float32),
        jax.ShapeDtypeStruct((E_PAD * NB,), jnp.float32),
        jax.ShapeDtypeStruct((TUP, D), jnp.float32),
    ],
    scratch_types=[
        pltpu.VMEM((XW,), jnp.int32),
        pltpu.VMEM((XW, D), jnp.float32),
        pltpu.VMEM((EW,), jnp.int32),
        pltpu.VMEM((EW,), jnp.int32),
        pltpu.VMEM((EW,), jnp.int32),
        pltpu.VMEM((EW,), jnp.int32),
        pltpu.VMEM((EW,), jnp.float32),
        pltpu.VMEM((EW,), jnp.int32),
        pltpu.VMEM((EW,), jnp.int32),
        [pltpu.VMEM((PREP_C,), jnp.float32) for _ in range(2)],
        [pltpu.VMEM((PREP_C,), jnp.float32) for _ in range(2)],
        pltpu.VMEM((NR * NB,), jnp.float32),
        pltpu.VMEM((NR * NB,), jnp.float32),
        [pltpu.VMEM((PREP_C * NB,), jnp.float32) for _ in range(2)],
        [pltpu.VMEM((PREP_C * NB,), jnp.float32) for _ in range(2)],
        pltpu.VMEM((TUP,), jnp.int32),
        pltpu.VMEM((TUP, D), jnp.float32),
        pltpu.SemaphoreType.DMA,
        pltpu.SemaphoreType.DMA,
        [pltpu.SemaphoreType.DMA for _ in range(2)],
        [pltpu.SemaphoreType.DMA for _ in range(2)],
    ],
)
def _sc_prep(entity_hbm, emb_hbm, gflat_hbm, ebi_hbm, st_hbm, et_hbm,
             type_hbm, norm_hbm, comp1_hbm, comp2_hbm, ut_hbm,
             x_out, w1_out, w2_out, temb_out,
             idx_v, rows_v, ebi_all, st_all, et_all, type_all, norm_all,
             ist_all, iet_all, gst, get, c1_v, c2_v, w1v, w2v, ut_v,
             trows_v, psem, xsem, gsem, osem):
    wid = _wid()
    esl = pl.ds(wid * EW, EW)

    cps = [pltpu.async_copy(ebi_hbm.at[esl], ebi_all, psem),
           pltpu.async_copy(st_hbm.at[esl], st_all, psem),
           pltpu.async_copy(et_hbm.at[esl], et_all, psem),
           pltpu.async_copy(type_hbm.at[esl], type_all, psem),
           pltpu.async_copy(norm_hbm.at[esl], norm_all, psem),
           pltpu.async_copy(comp1_hbm, c1_v, psem),
           pltpu.async_copy(comp2_hbm, c2_v, psem),
           pltpu.async_copy(entity_hbm.at[pl.ds(wid * XW, XW)], idx_v,
                            psem)]
    for cp in cps:
        cp.wait()

    pltpu.async_copy(emb_hbm.at[idx_v.at[pl.ds(0, 128)]],
                     rows_v.at[pl.ds(0, 128)], xsem)
    pltpu.async_copy(emb_hbm.at[idx_v.at[pl.ds(128, 128)]],
                     rows_v.at[pl.ds(128, 128)], xsem)
    pltpu.async_copy(emb_hbm.at[idx_v.at[pl.ds(256, 64)]],
                     rows_v.at[pl.ds(256, 64)], xsem)

    @pl.when(wid == 0)
    def _():
        pltpu.sync_copy(ut_hbm, ut_v)
        for k in range(TUP // 16):
            sl = pl.ds(k * 16, 16)
            ut_v[sl] = ut_v[sl] + NUM_ENT
        pltpu.async_copy(emb_hbm.at[ut_v], trows_v, psem).wait()
        pltpu.sync_copy(trows_v, temb_out)

    @plsc.parallel_loop(0, EW // 16, 1, unroll=4)
    def _idx_body(i):
        sl = pl.ds(i * 16, 16)
        bt = ebi_all[sl] * G_PAD
        ist_all[sl] = bt + st_all[sl]
        iet_all[sl] = bt + et_all[sl]

    def istsl(c):
        return ist_all.at[pl.ds(c * PREP_C, PREP_C)]

    def ietsl(c):
        return iet_all.at[pl.ds(c * PREP_C, PREP_C)]

    def w1sl(c):
        return w1_out.at[pl.ds((wid * EW + c * PREP_C) * NB, PREP_C * NB)]

    def w2sl(c):
        return w2_out.at[pl.ds((wid * EW + c * PREP_C) * NB, PREP_C * NB)]

    for j in range(2):
        pltpu.async_copy(gflat_hbm.at[istsl(j)], gst[j], gsem[j])
        pltpu.async_copy(gflat_hbm.at[ietsl(j)], get[j], gsem[j])

    lanes = lax.iota(jnp.int32, 16)

    def wchunk(c, b, last):
        pltpu.make_async_copy(gflat_hbm.at[istsl(c)], gst[b], gsem[b]).wait()
        pltpu.make_async_copy(gflat_hbm.at[ietsl(c)], get[b], gsem[b]).wait()
        @pl.when(c >= 2)
        def _():
            pltpu.make_async_copy(w1v[b], w1sl(c), osem[b]).wait()
            pltpu.make_async_copy(w2v[b], w2sl(c), osem[b]).wait()
        e0 = c * PREP_C
        for k in range(PREP_C // 16):
            sl = pl.ds(k * 16, 16)
            asl = pl.ds(e0 + k * 16, 16)
            z = (gst[b][sl] + get[b][sl]) * 0.5
            gate = 1.0 / (1.0 + jnp.exp(-z))
            sval = gate * norm_all[asl]
            tk = type_all[asl] * NB
            for bi in range(NB):
                cidx = lanes * NB + (k * 16 * NB + bi)
                tb = tk + bi
                plsc.store_scatter(w1v[b], [cidx],
                                   plsc.load_gather(c1_v, [tb]) * sval)
                plsc.store_scatter(w2v[b], [cidx],
                                   plsc.load_gather(c2_v, [tb]) * sval)
        pltpu.async_copy(w1v[b], w1sl(c), osem[b])
        pltpu.async_copy(w2v[b], w2sl(c), osem[b])
        if not last:
            pltpu.async_copy(gflat_hbm.at[istsl(c + 2)], gst[b], gsem[b])
            pltpu.async_copy(gflat_hbm.at[ietsl(c + 2)], get[b], gsem[b])

    def pair_body(q, _):
        wchunk(2 * q, 0, False)
        wchunk(2 * q + 1, 1, False)
        return 0

    lax.fori_loop(0, NCH_P // 2 - 1, pair_body, 0)
    wchunk(jnp.int32(NCH_P - 2), 0, True)
    wchunk(jnp.int32(NCH_P - 1), 1, True)
    for b in range(2):
        pltpu.make_async_copy(w1v[b], w1sl(jnp.int32(NCH_P - 2 + b)),
                              osem[b]).wait()
        pltpu.make_async_copy(w2v[b], w2sl(jnp.int32(NCH_P - 2 + b)),
                              osem[b]).wait()

    pltpu.make_async_copy(emb_hbm.at[idx_v.at[pl.ds(0, 128)]],
                          rows_v.at[pl.ds(0, 128)], xsem).wait()
    pltpu.make_async_copy(emb_hbm.at[idx_v.at[pl.ds(128, 128)]],
                          rows_v.at[pl.ds(128, 128)], xsem).wait()
    pltpu.make_async_copy(emb_hbm.at[idx_v.at[pl.ds(256, 64)]],
                          rows_v.at[pl.ds(256, 64)], xsem).wait()
    pltpu.sync_copy(rows_v, x_out.at[pl.ds(wid * XW, XW)])


@functools.partial(
    pl.kernel,
    mesh=_mesh,
    compiler_params=pltpu.CompilerParams(needs_layout_passes=False),
    out_type=jax.ShapeDtypeStruct((NC, N_ACC, D), jnp.float32),
    scratch_types=[
        pltpu.VMEM((EW,), jnp.int32),
        [pltpu.VMEM((LAY_C,), jnp.int32) for _ in range(4)],
        [pltpu.VMEM((LAY_C * NB,), jnp.float32) for _ in range(4)],
        [pltpu.VMEM((LAY_C, NB * D), jnp.float32) for _ in range(2)],
        [pltpu.VMEM((LAY_C, D), jnp.float32) for _ in range(2)],
        pltpu.VMEM_SHARED((N_ACC, D), jnp.float32),
        [pltpu.SemaphoreType.DMA for _ in range(2)],
        [pltpu.SemaphoreType.DMA for _ in range(4)],
        [pltpu.SemaphoreType.DMA for _ in range(2)],
        pltpu.SemaphoreType.DMA,
    ],
)
def _sc_layer(xb_hbm, src_hbm, dst_hbm, w_hbm, zr_hbm, part_out,
              sidx_all, didx, w, rows, msg, acc, gsem, wsem, ssem, psem):
    cid = lax.axis_index("c")
    sid = lax.axis_index("s")
    wid = cid * NS + sid
    nch = EW // LAY_C
    wstride = LAY_C * NB

    def wslice(ch):
        ch = jnp.minimum(ch, nch - 1)
        return w_hbm.at[pl.ds(wid * EW * NB + ch * wstride, wstride)]

    def dslice(ch):
        ch = jnp.minimum(ch, nch - 1)
        return dst_hbm.at[pl.ds(wid * EW + ch * LAY_C, LAY_C)]

    def sidx(ch):
        ch = jnp.minimum(ch, nch - 1)
        return sidx_all.at[pl.ds(ch * LAY_C, LAY_C)]

    c0 = pltpu.async_copy(src_hbm.at[pl.ds(wid * EW, EW)], sidx_all, psem)
    pltpu.sync_copy(zr_hbm, acc.at[pl.ds(sid * NROW, NROW)])
    c0.wait()
    plsc.subcore_barrier()

    for j in range(2):
        pltpu.async_copy(wslice(j), w[j], wsem[j])
        pltpu.async_copy(dslice(j), didx[j], wsem[j])
    pltpu.async_copy(xb_hbm.at[sidx(0)], rows[0], gsem[0])

    def compute(rows_v, w_v, msg_v):
        @plsc.parallel_loop(0, LAY_C, 1, unroll=8)
        def edge_body(c):
            cb = c * NB
            wb0 = plsc.load_gather(w_v, [_splat(cb)])
            wb1 = plsc.load_gather(w_v, [_splat(cb + 1)])
            wb2 = plsc.load_gather(w_v, [_splat(cb + 2)])
            wb3 = plsc.load_gather(w_v, [_splat(cb + 3)])
            for k in range(D // 16):
                o = k * 16
                m = wb0 * rows_v[c, pl.ds(o, 16)]
                m = m + wb1 * rows_v[c, pl.ds(D + o, 16)]
                m = m + wb2 * rows_v[c, pl.ds(2 * D + o, 16)]
                m = m + wb3 * rows_v[c, pl.ds(3 * D + o, 16)]
                msg_v[c, pl.ds(o, 16)] = m

    def half(c, j):
        mb = j % 2
        db = j % 4
        db2 = (j + 2) % 4
        pltpu.make_async_copy(xb_hbm.at[sidx(c)], rows[mb], gsem[mb]).wait()
        @pl.when(c >= 2)
        def _():
            pltpu.make_async_copy(msg[mb], acc.at[didx[db2]],
                                  ssem[mb]).wait()
        @pl.when(c + 1 < nch)
        def _():
            pltpu.async_copy(xb_hbm.at[sidx(c + 1)], rows[1 - mb],
                             gsem[1 - mb])
        @pl.when(c + 2 < nch)
        def _():
            pltpu.async_copy(wslice(c + 2), w[db2], wsem[db2])
            pltpu.async_copy(dslice(c + 2), didx[db2], wsem[db2])
        pltpu.make_async_copy(wslice(c), w[db], wsem[db]).wait()
        pltpu.make_async_copy(dslice(c), didx[db], wsem[db]).wait()
        compute(rows[mb], w[db], msg[mb])
        pltpu.async_copy(msg[mb], acc.at[didx[db]], ssem[mb], add=True)

    def quad_body(q, _):
        for j in range(4):
            half(4 * q + j, j)
        return 0

    lax.fori_loop(0, nch // 4, quad_body, 0)
    pltpu.make_async_copy(msg[0], acc.at[didx[2]], ssem[0]).wait()
    pltpu.make_async_copy(msg[1], acc.at[didx[3]], ssem[1]).wait()
    plsc.subcore_barrier()
    rsl = pl.ds(sid * NROW, NROW)
    pltpu.sync_copy(acc.at[rsl], part_out.at[cid, rsl])


EPW = 56
TPW = 24
PW = 16
BPW = B // PW


@functools.partial(
    pl.kernel,
    mesh=_mesh,
    compiler_params=pltpu.CompilerParams(needs_layout_passes=False),
    out_type=[
        jax.ShapeDtypeStruct((B, D), jnp.float32),
        jax.ShapeDtypeStruct((B, D), jnp.float32),
    ],
    scratch_types=[
        pltpu.VMEM((2 * EPW,), jnp.int32),
        pltpu.VMEM((2 * EPW,), jnp.int32),
        pltpu.VMEM((2 * EPW,), jnp.int32),
        pltpu.VMEM((2 * EPW,), jnp.int32),
        pltpu.VMEM((2 * EPW, D), jnp.float32),
        pltpu.VMEM((2 * EPW, D), jnp.float32),
        pltpu.VMEM((2 * EPW, D), jnp.float32),
        pltpu.VMEM((2 * EPW, D), jnp.float32),
        pltpu.VMEM((BPW * TPW,), jnp.int32),
        pltpu.VMEM((BPW * TPW, D), jnp.float32),
        pltpu.VMEM((BPW, D), jnp.float32),
        pltpu.VMEM((BPW, D), jnp.float32),
        pltpu.SemaphoreType.DMA,
    ],
)
def _sc_pool(h2_hbm, temb_hbm, epool_hbm, tpool_hbm, pe_out, pt_out,
             eidx_a, eidx_b, eidx_c, eidx_d, erows_a, erows_b, erows_c,
             erows_d, tidx_v, trows_v, pe_v, pt_v, sem):
    cid = lax.axis_index("c")
    sid = lax.axis_index("s")
    @pl.when(cid == 0)
    def _():
        wid = sid
        ne = BPW * EPW
        quarter = ne // 4
        eidx = [eidx_a, eidx_b, eidx_c, eidx_d]
        erows = [erows_a, erows_b, erows_c, erows_d]
        for qq in range(4):
            pltpu.sync_copy(
                epool_hbm.at[pl.ds(wid * ne + qq * quarter, quarter)],
                eidx[qq])
        for qq in range(4):
            pltpu.async_copy(h2_hbm.at[eidx[qq]], erows[qq], sem)
        for qq in range(4):
            pltpu.make_async_copy(h2_hbm.at[eidx[qq]], erows[qq],
                                  sem).wait()

        for j in range(BPW):
            ebuf = erows[j // 2]
            joff = (j % 2) * EPW

            def eadd(r, carry):
                return tuple(carry[k] + ebuf[joff + r, pl.ds(k * 16, 16)]
                             for k in range(D // 16))
            acc8 = lax.fori_loop(
                0, 50, eadd, tuple(jnp.zeros((16,), jnp.float32)
                                   for _ in range(D // 16)))
            for k in range(D // 16):
                pe_v[j, pl.ds(k * 16, 16)] = acc8[k] * (1.0 / 50.0)

        bsl = pl.ds(wid * BPW, BPW)
        pltpu.sync_copy(pe_v, pe_out.at[bsl])

    @pl.when(cid == 1)
    def _():
        wid = sid
        nt = BPW * TPW
        pltpu.sync_copy(tpool_hbm.at[pl.ds(wid * nt, nt)], tidx_v)
        pltpu.async_copy(temb_hbm.at[tidx_v], trows_v, sem).wait()

        for j in range(BPW):
            def tadd(r, carry):
                return tuple(carry[k] + trows_v[j * TPW + r,
                                                pl.ds(k * 16, 16)]
                             for k in range(D // 16))
            acc8t = lax.fori_loop(
                0, 20, tadd, tuple(jnp.zeros((16,), jnp.float32)
                                   for _ in range(D // 16)))
            for k in range(D // 16):
                pt_v[j, pl.ds(k * 16, 16)] = acc8t[k] * (1.0 / 20.0)

        bsl = pl.ds(wid * BPW, BPW)
        pltpu.sync_copy(pt_v, pt_out.at[bsl])


def _dot(a, b):
    return jax.lax.dot_general(a, b, (((1,), (0,)), ((), ())),
                               preferred_element_type=jnp.float32)


def _dot_t(a, b):
    return jax.lax.dot_general(a, b, (((1,), (1,)), ((), ())),
                               preferred_element_type=jnp.float32)


def _tc_prep_body(qb_ref, wl_ref, bl_ref, wa_ref, ba_ref, wp_ref, bp_ref,
                  q_ref, wq_ref, p_ref):
    qb = qb_ref[...]
    q_ref[...] = _dot(qb, wl_ref[...]) + bl_ref[...]
    wq_ref[...] = _dot(qb, wa_ref[...]) + ba_ref[...]
    p_ref[...] = jax.nn.sigmoid(_dot(qb, wp_ref[...]) + bp_ref[...])


def _tc_prep(qb, wl, bl, wa, ba, wpb, bpb):
    return pl.pallas_call(
        _tc_prep_body,
        out_shape=[jax.ShapeDtypeStruct((B, D), jnp.float32),
                   jax.ShapeDtypeStruct((B, D), jnp.float32),
                   jax.ShapeDtypeStruct((B, D), jnp.float32)],
    )(qb, wl, bl, wa, ba, wpb, bpb)


GT = 1280


def _tc_g_body(wq_ref, t_ref, g_ref):
    g_ref[...] = _dot_t(wq_ref[...], t_ref[...])


def _tc_g(wq, t):
    return pl.pallas_call(
        _tc_g_body,
        grid=(G_PAD // GT,),
        in_specs=[pl.BlockSpec((B, D), lambda i: (0, 0)),
                  pl.BlockSpec((GT, D), lambda i: (i, 0))],
        out_specs=pl.BlockSpec((B, GT), lambda i: (0, i)),
        out_shape=jax.ShapeDtypeStruct((B, G_PAD), jnp.float32),
    )(wq, t)


XT = 400


def _tc_xb_body(x_ref, bc_ref, xb_ref):
    xb_ref[...] = _dot(x_ref[...], bc_ref[...])


def _tc_xb(x, bcat):
    return pl.pallas_call(
        _tc_xb_body,
        grid=(N // XT,),
        in_specs=[pl.BlockSpec((XT, D), lambda i: (i, 0)),
                  pl.BlockSpec((D, NB * D), lambda i: (0, 0))],
        out_specs=pl.BlockSpec((XT, NB * D), lambda i: (i, 0)),
        out_shape=jax.ShapeDtypeStruct((N, NB * D), jnp.float32),
    )(x, bcat)


def _tc_combine_body(part_ref, x_ref, ws_ref, b_ref, bc_ref,
                     h_ref, xb_ref):
    h = part_ref[0] + part_ref[1] + _dot(x_ref[...], ws_ref[...]) \
        + b_ref[0:1, :]
    h = jnp.maximum(h, 0.0)
    h_ref[...] = h
    xb_ref[...] = _dot(h, bc_ref[...])


def _tc_combine(part, x, ws, bvec, bcat):
    return pl.pallas_call(
        _tc_combine_body,
        grid=(N // XT,),
        in_specs=[pl.BlockSpec((NC, XT, D), lambda i: (0, i, 0)),
                  pl.BlockSpec((XT, D), lambda i: (i, 0)),
                  pl.BlockSpec((D, D), lambda i: (0, 0)),
                  pl.BlockSpec((8, D), lambda i: (0, 0)),
                  pl.BlockSpec((D, NB * D), lambda i: (0, 0))],
        out_specs=[pl.BlockSpec((XT, D), lambda i: (i, 0)),
                   pl.BlockSpec((XT, NB * D), lambda i: (i, 0))],
        out_shape=[jax.ShapeDtypeStruct((N, D), jnp.float32),
                   jax.ShapeDtypeStruct((N, NB * D), jnp.float32)],
    )(part, x, ws, bvec, bcat)


def _tc_h2_body(part_ref, h_ref, ws_ref, b_ref, h2_ref):
    h2_ref[...] = part_ref[0] + part_ref[1] \
        + _dot(h_ref[...], ws_ref[...]) + b_ref[0:1, :]


def _tc_h2(part, h, ws, bvec):
    return pl.pallas_call(
        _tc_h2_body,
        grid=(N // XT,),
        in_specs=[pl.BlockSpec((NC, XT, D), lambda i: (0, i, 0)),
                  pl.BlockSpec((XT, D), lambda i: (i, 0)),
                  pl.BlockSpec((D, D), lambda i: (0, 0)),
                  pl.BlockSpec((8, D), lambda i: (0, 0))],
        out_specs=pl.BlockSpec((XT, D), lambda i: (i, 0)),
        out_shape=jax.ShapeDtypeStruct((N, D), jnp.float32),
    )(part, h, ws, bvec)


ST = 1280
NV = NUM_ENT + NUM_TIM


def _tc_score_body(q_ref, pe_ref, pt_ref, p_ref, emb_ref, out_ref):
    p = p_ref[...]
    pred = (q_ref[...] + pe_ref[...] * p + pt_ref[...] * (1.0 - p)) / 3.0
    nrm = jnp.sqrt(jnp.sum(pred * pred, axis=1, keepdims=True))
    pn = pred / (nrm + 1e-8)
    blk = emb_ref[...]
    nr = jnp.sqrt(jnp.sum(blk * blk, axis=1))
    s = _dot_t(pn, blk)
    out_ref[...] = s * (30.0 / (nr + 1e-8))[None, :]


def _tc_score(q, pe, pt, p, emb):
    bspec = pl.BlockSpec((B, D), lambda i: (0, 0))
    return pl.pallas_call(
        _tc_score_body,
        grid=((NV + ST - 1) // ST,),
        in_specs=[bspec, bspec, bspec, bspec,
                  pl.BlockSpec((ST, D), lambda i: (i, 0))],
        out_specs=pl.BlockSpec((B, ST), lambda i: (0, i)),
        out_shape=jax.ShapeDtypeStruct((B, NV), jnp.float32),
    )(q, pe, pt, p, emb)


def kernel(ques_emb_bert, entity, edge_index, edge_type, start_time,
           end_time, edge_norm, edge_batch_idx, uniq_times, ent_pool_idx,
           time_pool_idx, emb_table, basis1, comp1, w_self1, b1, basis2,
           comp2, w_self2, b2, w_lin, b_lin, w_attn, b_attn, w_p, b_p):
    i32 = jnp.int32
    entity = entity.astype(i32)
    edge_type = edge_type.astype(i32)
    start_time = start_time.astype(i32)
    end_time = end_time.astype(i32)
    edge_batch_idx = edge_batch_idx.astype(i32)
    uniq_times = uniq_times.astype(i32)

    wpb = jnp.broadcast_to(w_p, (768, D))
    bpb = jnp.broadcast_to(b_p, (D,))
    q_emb, wq, p = _tc_prep(ques_emb_bert, w_lin, b_lin, w_attn, b_attn,
                            wpb, bpb)
    tpad = jnp.concatenate(
        [emb_table[NUM_ENT:], jnp.zeros((G_PAD - NUM_TIM, D), jnp.float32)])
    g = _tc_g(wq, tpad)
    gflat = g.reshape(-1)

    pad_e = E_PAD - E
    src = jnp.concatenate([edge_index[0].astype(i32), jnp.zeros(pad_e, i32)])
    dst = jnp.concatenate([edge_index[1].astype(i32), jnp.zeros(pad_e, i32)])
    ebi_p = jnp.concatenate([edge_batch_idx, jnp.zeros(pad_e, i32)])
    st_p = jnp.concatenate([start_time, jnp.zeros(pad_e, i32)])
    et_p = jnp.concatenate([end_time, jnp.zeros(pad_e, i32)])
    ty_p = jnp.concatenate([edge_type, jnp.zeros(pad_e, i32)])
    nm_p = jnp.concatenate([edge_norm, jnp.zeros(pad_e, jnp.float32)])
    ent_p = jnp.concatenate([entity, jnp.zeros(N_PAD - N, i32)])
    ut_p = jnp.concatenate([uniq_times, jnp.zeros(TUP - TU, i32)])

    x_pad, w1, w2, t_emb = _sc_prep(ent_p, emb_table, gflat, ebi_p, st_p,
                                    et_p, ty_p, nm_p, comp1.reshape(-1),
                                    comp2.reshape(-1), ut_p)
    x = x_pad[:N]

    zr = jnp.zeros((NROW, D), jnp.float32)
    b1b = jnp.broadcast_to(b1, (8, D))
    b2b = jnp.broadcast_to(b2, (8, D))
    b1cat = jnp.transpose(basis1, (1, 0, 2)).reshape(D, NB * D)
    b2cat = jnp.transpose(basis2, (1, 0, 2)).reshape(D, NB * D)

    xb1 = _tc_xb(x, b1cat)
    part1 = _sc_layer(xb1, src, dst, w1, zr)
    h1, xb2 = _tc_combine(part1, x, w_self1, b1b, b2cat)
    part2 = _sc_layer(xb2, src, dst, w2, zr)
    h2 = _tc_h2(part2, h1, w_self2, b2b)

    ep = jnp.pad(ent_pool_idx.astype(i32), ((0, 0), (0, EPW - 50)))
    tp = jnp.pad(time_pool_idx.astype(i32), ((0, 0), (0, TPW - 20)))
    pe, pt = _sc_pool(h2, t_emb, ep.reshape(-1), tp.reshape(-1))

    return _tc_score(q_emb, pe, pt, p, emb_table)
